# baseline (speedup 1.0000x reference)
import functools

import jax
import jax.numpy as jnp
from jax import lax
from jax.experimental import pallas as pl
from jax.experimental.pallas import tpu as pltpu
from jax.experimental.pallas import tpu_sc as plsc

_N = 10000
_E = 320000
_D = 128
_H = 32
_C = 40

_NC = 2
_NS = 16
_NW = _NC * _NS
_B = 128
_G = 80
_EP = _NW * _G * _B
_NP = 10240
_RPT = _NP // _NS
_ZR = 128
_NB = 4
_DW = 16
_W1 = 2 * _H + 2 * _DW
_W2 = 2 * _H


def _agg_body(with_deg, y_hbm, src_hbm, dst_hbm, *refs):
    if with_deg:
        (out, src_v, dst_v, rows_v, zb_v, acc_sp, gsems, ssems,
         ones_v, zd_v, deg_sp, dsems) = refs
    else:
        (out, src_v, dst_v, rows_v, zb_v, acc_sp, gsems, ssems) = refs
    c = lax.axis_index("c")
    s = lax.axis_index("s")
    wid = c * _NS + s

    pltpu.sync_copy(src_hbm.at[wid], src_v)
    pltpu.sync_copy(dst_hbm.at[wid], dst_v)

    def zf(i, _):
        zb_v[i, pl.ds(0, 16)] = jnp.zeros((16,), jnp.float32)
        zb_v[i, pl.ds(16, 16)] = jnp.zeros((16,), jnp.float32)
        return 0

    lax.fori_loop(0, _ZR, zf, 0)
    row0 = s * _RPT
    for k in range(_RPT // _ZR):
        pltpu.sync_copy(zb_v, acc_sp.at[pl.ds(row0 + k * _ZR, _ZR)])
    if with_deg:
        def zf2(i, _):
            zd_v[i, :] = jnp.zeros((_DW,), jnp.float32)
            return 0

        lax.fori_loop(0, _ZR, zf2, 0)

        def of(i, _):
            ones_v[i, :] = jnp.ones((_DW,), jnp.float32)
            return 0

        lax.fori_loop(0, _B, of, 0)
        for k in range(_RPT // _ZR):
            pltpu.sync_copy(zd_v, deg_sp.at[pl.ds(row0 + k * _ZR, _ZR)])

    plsc.subcore_barrier()

    for b in range(_NB):
        pltpu.async_copy(y_hbm.at[src_v.at[b]], rows_v.at[b], gsems.at[b])

    def step(p, _):
        for b in range(_NB):
            j = _NB * p + b
            pltpu.make_async_copy(
                y_hbm.at[src_v.at[0]], rows_v.at[b], gsems.at[b]).wait()
            pltpu.async_copy(
                rows_v.at[b], acc_sp.at[dst_v.at[j]], ssems.at[b], add=True)
            if with_deg:
                pltpu.async_copy(
                    ones_v, deg_sp.at[dst_v.at[j]], dsems.at[b], add=True)
        for b in range(_NB):
            j = _NB * p + b
            pltpu.make_async_copy(
                rows_v.at[b], acc_sp.at[pl.ds(0, _B)], ssems.at[b]).wait()
            if with_deg:
                pltpu.make_async_copy(
                    ones_v, deg_sp.at[pl.ds(0, _B)], dsems.at[b]).wait()
            nxt = jnp.minimum(j + _NB, _G - 1)
            pltpu.async_copy(y_hbm.at[src_v.at[nxt]], rows_v.at[b], gsems.at[b])
        return 0

    lax.fori_loop(0, _G // _NB, step, 0)
    for b in range(_NB):
        pltpu.make_async_copy(
            y_hbm.at[src_v.at[0]], rows_v.at[b], gsems.at[b]).wait()
    plsc.subcore_barrier()

    pltpu.sync_copy(acc_sp.at[pl.ds(row0, _RPT)],
                    out.at[s, :, pl.ds(c * _H, _H)])
    if with_deg:
        pltpu.sync_copy(deg_sp.at[pl.ds(row0, _RPT)],
                        out.at[s, :, pl.ds(2 * _H + c * _DW, _DW)])


def _make_agg(with_deg):
    mesh = plsc.VectorSubcoreMesh(core_axis_name="c", subcore_axis_name="s")
    width = _W1 if with_deg else _W2
    out_type = jax.ShapeDtypeStruct((_NS, _RPT, width), jnp.float32)
    scratch = [
        pltpu.VMEM((_G, _B), jnp.int32),
        pltpu.VMEM((_G, _B), jnp.int32),
        pltpu.VMEM((_NB, _B, _H), jnp.float32),
        pltpu.VMEM((_ZR, _H), jnp.float32),
        pltpu.VMEM_SHARED((_NP, _H), jnp.float32),
        pltpu.SemaphoreType.DMA((_NB,)),
        pltpu.SemaphoreType.DMA((_NB,)),
    ]
    if with_deg:
        scratch += [
            pltpu.VMEM((_B, _DW), jnp.float32),
            pltpu.VMEM((_ZR, _DW), jnp.float32),
            pltpu.VMEM_SHARED((_NP, _DW), jnp.float32),
            pltpu.SemaphoreType.DMA((_NB,)),
        ]
    return pl.kernel(
        functools.partial(_agg_body, with_deg),
        out_type=out_type,
        mesh=mesh,
        scratch_types=scratch,
        compiler_params=pltpu.CompilerParams(use_tc_tiling_on_sc=False),
    )


_agg_deg = _make_agg(True)
_agg = _make_agg(False)


def _proj1_body(x_ref, wl_ref, wr_ref, bl_ref, y_ref, z_ref):
    xb = x_ref[...]
    dn = (((1,), (1,)), ((), ()))
    y_ref[...] = lax.dot_general(xb, wl_ref[...], dn,
                                 preferred_element_type=jnp.float32)
    z_ref[...] = lax.dot_general(xb, wr_ref[...], dn,
                                 preferred_element_type=jnp.float32) + bl_ref[...]


def _proj1(x, wl1, wr1, bl1):
    bn = 2000
    return pl.pallas_call(
        _proj1_body,
        grid=(_N // bn,),
        in_specs=[
            pl.BlockSpec((bn, _D), lambda i: (i, 0)),
            pl.BlockSpec((_H, _D), lambda i: (0, 0)),
            pl.BlockSpec((_H, _D), lambda i: (0, 0)),
            pl.BlockSpec((1, _H), lambda i: (0, 0)),
        ],
        out_specs=[
            pl.BlockSpec((bn, _H), lambda i: (i, 0)),
            pl.BlockSpec((bn, _H), lambda i: (i, 0)),
        ],
        out_shape=[
            jax.ShapeDtypeStruct((_NP, _H), jnp.float32),
            jax.ShapeDtypeStruct((_NP, _H), jnp.float32),
        ],
    )(x, wl1, wr1, bl1)


def _mid_body(f_ref, z_ref, h_ref, dinv_ref):
    f = f_ref[...]
    ssum = f[:, 0:_H] + f[:, _H:2 * _H]
    deg = f[:, 2 * _H:2 * _H + 1] + f[:, 2 * _H + _DW:2 * _H + _DW + 1]
    dinv = 1.0 / jnp.maximum(deg, 1.0)
    h_ref[...] = jnp.maximum(ssum * dinv + z_ref[...], 0.0)
    dinv_ref[...] = dinv


def _mid(f1, z1):
    bn = 2048
    return pl.pallas_call(
        _mid_body,
        grid=(_NP // bn,),
        in_specs=[
            pl.BlockSpec((bn, _W1), lambda i: (i, 0)),
            pl.BlockSpec((bn, _H), lambda i: (i, 0)),
        ],
        out_specs=[
            pl.BlockSpec((bn, _H), lambda i: (i, 0)),
            pl.BlockSpec((bn, 1), lambda i: (i, 0)),
        ],
        out_shape=[
            jax.ShapeDtypeStruct((_NP, _H), jnp.float32),
            jax.ShapeDtypeStruct((_NP, 1), jnp.float32),
        ],
    )(f1, z1)


def _final_body(f_ref, dinv_ref, h_ref, wl2_ref, bl2_ref, wr2_ref, o_ref):
    f = f_ref[...]
    mean2 = (f[:, 0:_H] + f[:, _H:2 * _H]) * dinv_ref[...]
    dn = (((1,), (1,)), ((), ()))
    t = (lax.dot_general(mean2, wl2_ref[...], dn,
                         preferred_element_type=jnp.float32)
         + bl2_ref[...]
         + lax.dot_general(h_ref[...], wr2_ref[...], dn,
                           preferred_element_type=jnp.float32))
    m = jnp.max(t, axis=1, keepdims=True)
    lse = jnp.log(jnp.sum(jnp.exp(t - m), axis=1, keepdims=True))
    o_ref[...] = t - m - lse


def _final(f2, dinv, h, wl2, bl2, wr2):
    bn = 2000
    return pl.pallas_call(
        _final_body,
        grid=(_N // bn,),
        in_specs=[
            pl.BlockSpec((bn, _W2), lambda i: (i, 0)),
            pl.BlockSpec((bn, 1), lambda i: (i, 0)),
            pl.BlockSpec((bn, _H), lambda i: (i, 0)),
            pl.BlockSpec((_C, _H), lambda i: (0, 0)),
            pl.BlockSpec((1, _C), lambda i: (0, 0)),
            pl.BlockSpec((_C, _H), lambda i: (0, 0)),
        ],
        out_specs=pl.BlockSpec((bn, _C), lambda i: (i, 0)),
        out_shape=jax.ShapeDtypeStruct((_N, _C), jnp.float32),
    )(f2, dinv, h, wl2, bl2, wr2)


def kernel(x, edge_index, Wl1, bl1, Wr1, Wl2, bl2, Wr2):
    ppt = (_EP - _E) // _NW
    dummy_src = jnp.zeros((_NW, ppt), jnp.int32)
    dummy_dst = jnp.broadcast_to(
        _N + jnp.arange(ppt, dtype=jnp.int32), (_NW, ppt))
    src = jnp.concatenate(
        [edge_index[0].reshape(_NW, _E // _NW), dummy_src],
        axis=1).reshape(_NW, _G, _B)
    dst = jnp.concatenate(
        [edge_index[1].reshape(_NW, _E // _NW), dummy_dst],
        axis=1).reshape(_NW, _G, _B)
    y1, z1 = _proj1(x, Wl1, Wr1, bl1.reshape(1, _H))
    f1 = _agg_deg(y1, src, dst).reshape(_NP, _W1)
    h, dinv = _mid(f1, z1)
    f2 = _agg(h, src, dst).reshape(_NP, _W2)
    out = _final(f2, dinv, h, Wl2, bl2.reshape(1, _C), Wr2)
    return out


# speedup vs baseline: 3.5044x; 3.5044x over previous
import functools

import jax
import jax.numpy as jnp
from jax import lax
from jax.experimental import pallas as pl
from jax.experimental.pallas import tpu as pltpu
from jax.experimental.pallas import tpu_sc as plsc

_N = 10000
_E = 320000
_D = 128
_H = 32
_C = 40

_NC = 2
_NS = 16
_NW = _NC * _NS
_B = 125
_G = 80
_NP = 10240
_RPT = _NP // _NS
_ZR = 128
_NB = 4
_DW = 16
_W1 = 2 * _H + 2 * _DW
_W2 = 2 * _H


def _agg_body(with_deg, y_hbm, src_hbm, dst_hbm, *refs):
    if with_deg:
        (out, src_v, dst_v, rows_v, zb_v, acc_sp, gsems, ssems,
         ones_v, zd_v, deg_sp, dsems) = refs
    else:
        (out, src_v, dst_v, rows_v, zb_v, acc_sp, gsems, ssems) = refs
    c = lax.axis_index("c")
    s = lax.axis_index("s")
    wid = c * _NS + s

    pltpu.sync_copy(src_hbm.at[wid], src_v)
    pltpu.sync_copy(dst_hbm.at[wid], dst_v)

    def zf(i, _):
        zb_v[i, pl.ds(0, 16)] = jnp.zeros((16,), jnp.float32)
        zb_v[i, pl.ds(16, 16)] = jnp.zeros((16,), jnp.float32)
        return 0

    lax.fori_loop(0, _ZR, zf, 0)
    row0 = s * _RPT
    for k in range(_RPT // _ZR):
        pltpu.sync_copy(zb_v, acc_sp.at[pl.ds(row0 + k * _ZR, _ZR)])
    if with_deg:
        def zf2(i, _):
            zd_v[i, :] = jnp.zeros((_DW,), jnp.float32)
            return 0

        lax.fori_loop(0, _ZR, zf2, 0)

        def of(i, _):
            ones_v[i, :] = jnp.ones((_DW,), jnp.float32)
            return 0

        lax.fori_loop(0, _B, of, 0)
        for k in range(_RPT // _ZR):
            pltpu.sync_copy(zd_v, deg_sp.at[pl.ds(row0 + k * _ZR, _ZR)])

    plsc.subcore_barrier()

    for b in range(_NB):
        pltpu.async_copy(y_hbm.at[src_v.at[b]], rows_v.at[b], gsems.at[b])

    def step(p, _):
        for b in range(_NB):
            j = _NB * p + b
            pltpu.make_async_copy(
                y_hbm.at[src_v.at[0]], rows_v.at[b], gsems.at[b]).wait()
            pltpu.async_copy(
                rows_v.at[b], acc_sp.at[dst_v.at[j]], ssems.at[b], add=True)
            if with_deg:
                pltpu.async_copy(
                    ones_v, deg_sp.at[dst_v.at[j]], dsems.at[b], add=True)
        for b in range(_NB):
            j = _NB * p + b
            pltpu.make_async_copy(
                rows_v.at[b], acc_sp.at[pl.ds(0, _B)], ssems.at[b]).wait()
            if with_deg:
                pltpu.make_async_copy(
                    ones_v, deg_sp.at[pl.ds(0, _B)], dsems.at[b]).wait()
            nxt = jnp.minimum(j + _NB, _G - 1)
            pltpu.async_copy(y_hbm.at[src_v.at[nxt]], rows_v.at[b], gsems.at[b])
        return 0

    lax.fori_loop(0, _G // _NB, step, 0)
    for b in range(_NB):
        pltpu.make_async_copy(
            y_hbm.at[src_v.at[0]], rows_v.at[b], gsems.at[b]).wait()
    plsc.subcore_barrier()

    pltpu.sync_copy(acc_sp.at[pl.ds(row0, _RPT)],
                    out.at[s, :, pl.ds(c * _H, _H)])
    if with_deg:
        pltpu.sync_copy(deg_sp.at[pl.ds(row0, _RPT)],
                        out.at[s, :, pl.ds(2 * _H + c * _DW, _DW)])


def _make_agg(with_deg):
    mesh = plsc.VectorSubcoreMesh(core_axis_name="c", subcore_axis_name="s")
    width = _W1 if with_deg else _W2
    out_type = jax.ShapeDtypeStruct((_NS, _RPT, width), jnp.float32)
    scratch = [
        pltpu.VMEM((_G, _B), jnp.int32),
        pltpu.VMEM((_G, _B), jnp.int32),
        pltpu.VMEM((_NB, _B, _H), jnp.float32),
        pltpu.VMEM((_ZR, _H), jnp.float32),
        pltpu.VMEM_SHARED((_NP, _H), jnp.float32),
        pltpu.SemaphoreType.DMA((_NB,)),
        pltpu.SemaphoreType.DMA((_NB,)),
    ]
    if with_deg:
        scratch += [
            pltpu.VMEM((_B, _DW), jnp.float32),
            pltpu.VMEM((_ZR, _DW), jnp.float32),
            pltpu.VMEM_SHARED((_NP, _DW), jnp.float32),
            pltpu.SemaphoreType.DMA((_NB,)),
        ]
    return pl.kernel(
        functools.partial(_agg_body, with_deg),
        out_type=out_type,
        mesh=mesh,
        scratch_types=scratch,
        compiler_params=pltpu.CompilerParams(use_tc_tiling_on_sc=False),
    )


_agg_deg = _make_agg(True)
_agg = _make_agg(False)


def _proj1_body(x_ref, wl_ref, wr_ref, bl_ref, y_ref, z_ref):
    xb = x_ref[...]
    dn = (((1,), (1,)), ((), ()))
    y_ref[...] = lax.dot_general(xb, wl_ref[...], dn,
                                 preferred_element_type=jnp.float32)
    z_ref[...] = lax.dot_general(xb, wr_ref[...], dn,
                                 preferred_element_type=jnp.float32) + bl_ref[...]


def _proj1(x, wl1, wr1, bl1):
    bn = 2000
    return pl.pallas_call(
        _proj1_body,
        grid=(_N // bn,),
        in_specs=[
            pl.BlockSpec((bn, _D), lambda i: (i, 0)),
            pl.BlockSpec((_H, _D), lambda i: (0, 0)),
            pl.BlockSpec((_H, _D), lambda i: (0, 0)),
            pl.BlockSpec((1, _H), lambda i: (0, 0)),
        ],
        out_specs=[
            pl.BlockSpec((bn, _H), lambda i: (i, 0)),
            pl.BlockSpec((bn, _H), lambda i: (i, 0)),
        ],
        out_shape=[
            jax.ShapeDtypeStruct((_NP, _H), jnp.float32),
            jax.ShapeDtypeStruct((_NP, _H), jnp.float32),
        ],
    )(x, wl1, wr1, bl1)


def _mid_body(f_ref, z_ref, h_ref, dinv_ref):
    f = f_ref[...]
    ssum = f[:, 0:_H] + f[:, _H:2 * _H]
    deg = f[:, 2 * _H:2 * _H + 1] + f[:, 2 * _H + _DW:2 * _H + _DW + 1]
    dinv = 1.0 / jnp.maximum(deg, 1.0)
    h_ref[...] = jnp.maximum(ssum * dinv + z_ref[...], 0.0)
    dinv_ref[...] = dinv


def _mid(f1, z1):
    bn = 2048
    return pl.pallas_call(
        _mid_body,
        grid=(_NP // bn,),
        in_specs=[
            pl.BlockSpec((bn, _W1), lambda i: (i, 0)),
            pl.BlockSpec((bn, _H), lambda i: (i, 0)),
        ],
        out_specs=[
            pl.BlockSpec((bn, _H), lambda i: (i, 0)),
            pl.BlockSpec((bn, 1), lambda i: (i, 0)),
        ],
        out_shape=[
            jax.ShapeDtypeStruct((_NP, _H), jnp.float32),
            jax.ShapeDtypeStruct((_NP, 1), jnp.float32),
        ],
    )(f1, z1)


def _final_body(f_ref, dinv_ref, h_ref, wl2_ref, bl2_ref, wr2_ref, o_ref):
    f = f_ref[...]
    mean2 = (f[:, 0:_H] + f[:, _H:2 * _H]) * dinv_ref[...]
    dn = (((1,), (1,)), ((), ()))
    t = (lax.dot_general(mean2, wl2_ref[...], dn,
                         preferred_element_type=jnp.float32)
         + bl2_ref[...]
         + lax.dot_general(h_ref[...], wr2_ref[...], dn,
                           preferred_element_type=jnp.float32))
    m = jnp.max(t, axis=1, keepdims=True)
    lse = jnp.log(jnp.sum(jnp.exp(t - m), axis=1, keepdims=True))
    o_ref[...] = t - m - lse


def _final(f2, dinv, h, wl2, bl2, wr2):
    bn = 2000
    return pl.pallas_call(
        _final_body,
        grid=(_N // bn,),
        in_specs=[
            pl.BlockSpec((bn, _W2), lambda i: (i, 0)),
            pl.BlockSpec((bn, 1), lambda i: (i, 0)),
            pl.BlockSpec((bn, _H), lambda i: (i, 0)),
            pl.BlockSpec((_C, _H), lambda i: (0, 0)),
            pl.BlockSpec((1, _C), lambda i: (0, 0)),
            pl.BlockSpec((_C, _H), lambda i: (0, 0)),
        ],
        out_specs=pl.BlockSpec((bn, _C), lambda i: (i, 0)),
        out_shape=jax.ShapeDtypeStruct((_N, _C), jnp.float32),
    )(f2, dinv, h, wl2, bl2, wr2)


def kernel(x, edge_index, Wl1, bl1, Wr1, Wl2, bl2, Wr2):
    src = edge_index[0].reshape(_NW, _G, _B)
    dst = edge_index[1].reshape(_NW, _G, _B)
    y1, z1 = _proj1(x, Wl1, Wr1, bl1.reshape(1, _H))
    f1 = _agg_deg(y1, src, dst).reshape(_NP, _W1)
    h, dinv = _mid(f1, z1)
    f2 = _agg(h, src, dst).reshape(_NP, _W2)
    out = _final(f2, dinv, h, Wl2, bl2.reshape(1, _C), Wr2)
    return out


# speedup vs baseline: 3.8943x; 1.1113x over previous
import functools

import jax
import jax.numpy as jnp
from jax import lax
from jax.experimental import pallas as pl
from jax.experimental.pallas import tpu as pltpu
from jax.experimental.pallas import tpu_sc as plsc

_N = 10000
_E = 320000
_D = 128
_H = 32
_C = 40

_NC = 2
_NS = 16
_NW = _NC * _NS
_B = 125
_G = 80
_NP = 10240
_RPT = _NP // _NS
_ZR = 128
_NB = 8
_DW = 16
_W1 = 2 * _H + 2 * _DW
_W2 = 2 * _H


def _agg_body(with_deg, y_hbm, ei_hbm, *refs):
    if with_deg:
        (out, src_v, dst_v, rows_v, zb_v, acc_sp, gsems, ssems,
         ones_v, zd_v, deg_sp, dsems) = refs
    else:
        (out, src_v, dst_v, rows_v, zb_v, acc_sp, gsems, ssems) = refs
    c = lax.axis_index("c")
    s = lax.axis_index("s")
    wid = c * _NS + s

    pltpu.sync_copy(ei_hbm.at[0, wid], src_v)
    pltpu.sync_copy(ei_hbm.at[1, wid], dst_v)

    def zf(i, _):
        zb_v[i, pl.ds(0, 16)] = jnp.zeros((16,), jnp.float32)
        zb_v[i, pl.ds(16, 16)] = jnp.zeros((16,), jnp.float32)
        return 0

    lax.fori_loop(0, _ZR, zf, 0)
    row0 = s * _RPT
    for k in range(_RPT // _ZR):
        pltpu.sync_copy(zb_v, acc_sp.at[pl.ds(row0 + k * _ZR, _ZR)])
    if with_deg:
        def zf2(i, _):
            zd_v[i, :] = jnp.zeros((_DW,), jnp.float32)
            return 0

        lax.fori_loop(0, _ZR, zf2, 0)

        def of(i, _):
            ones_v[i, :] = jnp.ones((_DW,), jnp.float32)
            return 0

        lax.fori_loop(0, _B, of, 0)
        for k in range(_RPT // _ZR):
            pltpu.sync_copy(zd_v, deg_sp.at[pl.ds(row0 + k * _ZR, _ZR)])

    plsc.subcore_barrier()

    for b in range(_NB):
        pltpu.async_copy(y_hbm.at[src_v.at[b]], rows_v.at[b], gsems.at[b])

    def step(p, _):
        for b in range(_NB):
            j = _NB * p + b
            pltpu.make_async_copy(
                y_hbm.at[src_v.at[0]], rows_v.at[b], gsems.at[b]).wait()
            pltpu.async_copy(
                rows_v.at[b], acc_sp.at[dst_v.at[j]], ssems.at[b], add=True)
            if with_deg:
                pltpu.async_copy(
                    ones_v, deg_sp.at[dst_v.at[j]], dsems.at[b], add=True)
        for b in range(_NB):
            j = _NB * p + b
            pltpu.make_async_copy(
                rows_v.at[b], acc_sp.at[pl.ds(0, _B)], ssems.at[b]).wait()
            if with_deg:
                pltpu.make_async_copy(
                    ones_v, deg_sp.at[pl.ds(0, _B)], dsems.at[b]).wait()
            nxt = jnp.minimum(j + _NB, _G - 1)
            pltpu.async_copy(y_hbm.at[src_v.at[nxt]], rows_v.at[b], gsems.at[b])
        return 0

    lax.fori_loop(0, _G // _NB, step, 0)
    for b in range(_NB):
        pltpu.make_async_copy(
            y_hbm.at[src_v.at[0]], rows_v.at[b], gsems.at[b]).wait()
    plsc.subcore_barrier()

    pltpu.sync_copy(acc_sp.at[pl.ds(row0, _RPT)],
                    out.at[s, :, pl.ds(c * _H, _H)])
    if with_deg:
        pltpu.sync_copy(deg_sp.at[pl.ds(row0, _RPT)],
                        out.at[s, :, pl.ds(2 * _H + c * _DW, _DW)])


def _make_agg(with_deg):
    mesh = plsc.VectorSubcoreMesh(core_axis_name="c", subcore_axis_name="s")
    width = _W1 if with_deg else _W2
    out_type = jax.ShapeDtypeStruct((_NS, _RPT, width), jnp.float32)
    scratch = [
        pltpu.VMEM((_G, _B), jnp.int32),
        pltpu.VMEM((_G, _B), jnp.int32),
        pltpu.VMEM((_NB, _B, _H), jnp.float32),
        pltpu.VMEM((_ZR, _H), jnp.float32),
        pltpu.VMEM_SHARED((_NP, _H), jnp.float32),
        pltpu.SemaphoreType.DMA((_NB,)),
        pltpu.SemaphoreType.DMA((_NB,)),
    ]
    if with_deg:
        scratch += [
            pltpu.VMEM((_B, _DW), jnp.float32),
            pltpu.VMEM((_ZR, _DW), jnp.float32),
            pltpu.VMEM_SHARED((_NP, _DW), jnp.float32),
            pltpu.SemaphoreType.DMA((_NB,)),
        ]
    return pl.kernel(
        functools.partial(_agg_body, with_deg),
        out_type=out_type,
        mesh=mesh,
        scratch_types=scratch,
        compiler_params=pltpu.CompilerParams(use_tc_tiling_on_sc=False),
    )


_agg_deg = _make_agg(True)
_agg = _make_agg(False)


def _proj1_body(x_ref, wl_ref, wr_ref, bl_ref, y_ref, z_ref):
    xb = x_ref[...]
    dn = (((1,), (1,)), ((), ()))
    y_ref[...] = lax.dot_general(xb, wl_ref[...], dn,
                                 preferred_element_type=jnp.float32)
    z_ref[...] = lax.dot_general(xb, wr_ref[...], dn,
                                 preferred_element_type=jnp.float32) + bl_ref[...]


def _proj1(x, wl1, wr1, bl1):
    bn = 2000
    return pl.pallas_call(
        _proj1_body,
        grid=(_N // bn,),
        in_specs=[
            pl.BlockSpec((bn, _D), lambda i: (i, 0)),
            pl.BlockSpec((_H, _D), lambda i: (0, 0)),
            pl.BlockSpec((_H, _D), lambda i: (0, 0)),
            pl.BlockSpec((1, _H), lambda i: (0, 0)),
        ],
        out_specs=[
            pl.BlockSpec((bn, _H), lambda i: (i, 0)),
            pl.BlockSpec((bn, _H), lambda i: (i, 0)),
        ],
        out_shape=[
            jax.ShapeDtypeStruct((_NP, _H), jnp.float32),
            jax.ShapeDtypeStruct((_NP, _H), jnp.float32),
        ],
    )(x, wl1, wr1, bl1)


def _mid_body(f_ref, z_ref, h_ref):
    f = f_ref[...]
    ssum = f[:, 0:_H] + f[:, _H:2 * _H]
    deg = f[:, 2 * _H:2 * _H + 1] + f[:, 2 * _H + _DW:2 * _H + _DW + 1]
    dinv = 1.0 / jnp.maximum(deg, 1.0)
    h_ref[...] = jnp.maximum(ssum * dinv + z_ref[...], 0.0)


def _mid(f1, z1):
    bn = 2048
    return pl.pallas_call(
        _mid_body,
        grid=(_NP // bn,),
        in_specs=[
            pl.BlockSpec((bn, _W1), lambda i: (i, 0)),
            pl.BlockSpec((bn, _H), lambda i: (i, 0)),
        ],
        out_specs=pl.BlockSpec((bn, _H), lambda i: (i, 0)),
        out_shape=jax.ShapeDtypeStruct((_NP, _H), jnp.float32),
    )(f1, z1)


def _final_body(f_ref, f1_ref, h_ref, wl2_ref, bl2_ref, wr2_ref, o_ref):
    f = f_ref[...]
    f1 = f1_ref[...]
    deg = f1[:, 2 * _H:2 * _H + 1] + f1[:, 2 * _H + _DW:2 * _H + _DW + 1]
    dinv = 1.0 / jnp.maximum(deg, 1.0)
    mean2 = (f[:, 0:_H] + f[:, _H:2 * _H]) * dinv
    dn = (((1,), (1,)), ((), ()))
    t = (lax.dot_general(mean2, wl2_ref[...], dn,
                         preferred_element_type=jnp.float32)
         + bl2_ref[...]
         + lax.dot_general(h_ref[...], wr2_ref[...], dn,
                           preferred_element_type=jnp.float32))
    m = jnp.max(t, axis=1, keepdims=True)
    lse = jnp.log(jnp.sum(jnp.exp(t - m), axis=1, keepdims=True))
    o_ref[...] = t - m - lse


def _final(f2, f1, h, wl2, bl2, wr2):
    bn = 2000
    return pl.pallas_call(
        _final_body,
        grid=(_N // bn,),
        in_specs=[
            pl.BlockSpec((bn, _W2), lambda i: (i, 0)),
            pl.BlockSpec((bn, _W1), lambda i: (i, 0)),
            pl.BlockSpec((bn, _H), lambda i: (i, 0)),
            pl.BlockSpec((_C, _H), lambda i: (0, 0)),
            pl.BlockSpec((1, _C), lambda i: (0, 0)),
            pl.BlockSpec((_C, _H), lambda i: (0, 0)),
        ],
        out_specs=pl.BlockSpec((bn, _C), lambda i: (i, 0)),
        out_shape=jax.ShapeDtypeStruct((_N, _C), jnp.float32),
    )(f2, f1, h, wl2, bl2, wr2)


def kernel(x, edge_index, Wl1, bl1, Wr1, Wl2, bl2, Wr2):
    ei = edge_index.reshape(2, _NW, _G, _B)
    y1, z1 = _proj1(x, Wl1, Wr1, bl1.reshape(1, _H))
    f1 = _agg_deg(y1, ei).reshape(_NP, _W1)
    h = _mid(f1, z1)
    f2 = _agg(h, ei).reshape(_NP, _W2)
    out = _final(f2, f1, h, Wl2, bl2.reshape(1, _C), Wr2)
    return out
